# Initial kernel scaffold; baseline (speedup 1.0000x reference)
import jax
import jax.numpy as jnp
from jax.experimental import pallas as pl


def kernel(x, edge_index, batch, W1, b1, W2, b2, W3, b3, W4, b4, Wl, bl):
    raise NotImplementedError("write your pallas kernel here")


# speedup vs baseline: 6.0870x; 6.0870x over previous
import functools

import jax
import jax.numpy as jnp
from jax import lax
from jax.experimental import pallas as pl
from jax.experimental.pallas import tpu as pltpu
from jax.experimental.pallas import tpu_sc as plsc

N = 10000
D = 128
DH = D // 2
E = 320000
G = 128
NCLS = 10

NC, NS = 2, 16
NW = NC * NS
EPW = E // NW
EPC = E // NS
C = 80
NCHUNK_DEG = EPW // C
NCHUNK = EPC // C

NP = 10240
NPS = NP // NS
DW = 16

_mesh = plsc.VectorSubcoreMesh(core_axis_name="c", subcore_axis_name="s")


@functools.partial(
    pl.kernel,
    out_type=jax.ShapeDtypeStruct((NC * NP, DW), jnp.float32),
    mesh=_mesh,
    scratch_types=[
        pltpu.VMEM((C,), jnp.int32),
        pltpu.VMEM((C, DW), jnp.float32),
        pltpu.VMEM((NPS, DW), jnp.float32),
        pltpu.VMEM_SHARED((NP, DW), jnp.float32),
    ],
    compiler_params=pltpu.CompilerParams(use_tc_tiling_on_sc=False),
)
def _sc_degree(dst_hbm, zeros_hbm, ones_hbm, out_hbm, didx, ones, bounce, acc):
    c = lax.axis_index("c")
    s = lax.axis_index("s")
    wid = s * NC + c
    pltpu.sync_copy(ones_hbm, ones)
    pltpu.sync_copy(zeros_hbm, bounce)
    pltpu.sync_copy(bounce, acc.at[pl.ds(s * NPS, NPS)])
    plsc.subcore_barrier()
    ebase = wid * EPW

    def body(i, carry):
        b = ebase + i * C
        pltpu.sync_copy(dst_hbm.at[pl.ds(b, C)], didx)
        pltpu.sync_copy(ones, acc.at[didx], add=True)
        return carry

    lax.fori_loop(0, NCHUNK_DEG, body, 0)
    plsc.subcore_barrier()
    pltpu.sync_copy(acc.at[pl.ds(s * NPS, NPS)], bounce)
    pltpu.sync_copy(bounce, out_hbm.at[pl.ds(c * NP + s * NPS, NPS)])


@functools.partial(
    pl.kernel,
    out_type=jax.ShapeDtypeStruct((NC * NP, DH), jnp.float32),
    mesh=_mesh,
    scratch_types=[
        pltpu.VMEM((C,), jnp.int32),
        pltpu.VMEM((C,), jnp.int32),
        pltpu.VMEM((C, DH), jnp.float32),
        pltpu.VMEM((NPS, DH), jnp.float32),
        pltpu.VMEM_SHARED((NP, DH), jnp.float32),
        pltpu.SemaphoreType.DMA,
    ],
    compiler_params=pltpu.CompilerParams(use_tc_tiling_on_sc=False),
)
def _sc_prop(u_hbm, src2_hbm, dst_hbm, zeros_hbm, out_hbm,
             sidx, didx, rows, bounce, acc, sem):
    c = lax.axis_index("c")
    s = lax.axis_index("s")
    pltpu.sync_copy(zeros_hbm, bounce)
    pltpu.sync_copy(bounce, acc.at[pl.ds(s * NPS, NPS)])
    plsc.subcore_barrier()
    sbase = c * E + s * EPC
    dbase = s * EPC

    def body(i, carry):
        o = i * C
        pltpu.sync_copy(src2_hbm.at[pl.ds(sbase + o, C)], sidx)
        pltpu.sync_copy(dst_hbm.at[pl.ds(dbase + o, C)], didx)
        pltpu.async_copy(u_hbm.at[sidx], rows, sem).wait()
        pltpu.sync_copy(rows, acc.at[didx], add=True)
        return carry

    lax.fori_loop(0, NCHUNK, body, 0)
    plsc.subcore_barrier()
    pltpu.sync_copy(acc.at[pl.ds(s * NPS, NPS)], bounce)
    pltpu.sync_copy(bounce, out_hbm.at[pl.ds(c * NP + s * NPS, NPS)])


R = 400
NBLK = N // R


def _split(u):
    return u[:, :DH], u[:, DH:]


def _tc0_body(x_ref, w_ref, deg_ref, u_ref, dis_ref):
    d = deg_ref[:, 0:1] + deg_ref[:, 1:2] + 1.0
    dis = lax.rsqrt(d)
    dis_ref[...] = dis
    u = jnp.dot(x_ref[...], w_ref[...],
                preferred_element_type=jnp.float32) * dis
    u_ref[0], u_ref[1] = _split(u)


_tc0 = pl.pallas_call(
    _tc0_body,
    grid=(NBLK,),
    in_specs=[
        pl.BlockSpec((R, D), lambda i: (i, 0)),
        pl.BlockSpec((D, D), lambda i: (0, 0)),
        pl.BlockSpec((R, 2), lambda i: (i, 0)),
    ],
    out_specs=[
        pl.BlockSpec((2, R, DH), lambda i: (0, i, 0)),
        pl.BlockSpec((R, 1), lambda i: (i, 0)),
    ],
    out_shape=[
        jax.ShapeDtypeStruct((2, N, DH), jnp.float32),
        jax.ShapeDtypeStruct((N, 1), jnp.float32),
    ],
)


def _tc_mid_body(a_ref, u_ref, dis_ref, b_ref, w_ref, h_ref, un_ref):
    dis = dis_ref[...]
    agg = jnp.concatenate([a_ref[0] + u_ref[0], a_ref[1] + u_ref[1]], axis=1)
    h = jnp.maximum(agg * dis + b_ref[...], 0.0)
    h_ref[...] = h
    un = jnp.dot(h, w_ref[...], preferred_element_type=jnp.float32) * dis
    un_ref[0], un_ref[1] = _split(un)


_tc_mid = pl.pallas_call(
    _tc_mid_body,
    grid=(NBLK,),
    in_specs=[
        pl.BlockSpec((NC, R, DH), lambda i: (0, i, 0)),
        pl.BlockSpec((2, R, DH), lambda i: (0, i, 0)),
        pl.BlockSpec((R, 1), lambda i: (i, 0)),
        pl.BlockSpec((1, D), lambda i: (0, 0)),
        pl.BlockSpec((D, D), lambda i: (0, 0)),
    ],
    out_specs=[
        pl.BlockSpec((R, D), lambda i: (i, 0)),
        pl.BlockSpec((2, R, DH), lambda i: (0, i, 0)),
    ],
    out_shape=[
        jax.ShapeDtypeStruct((N, D), jnp.float32),
        jax.ShapeDtypeStruct((2, N, DH), jnp.float32),
    ],
)


def _tc_final_body(a_ref, u_ref, dis_ref, b_ref, batch_ref, wl_ref, bl_ref,
                   h_ref, z_ref, p_scr):
    i = pl.program_id(0)
    agg = jnp.concatenate([a_ref[0] + u_ref[0], a_ref[1] + u_ref[1]], axis=1)
    h = jnp.maximum(agg * dis_ref[...] + b_ref[...], 0.0)
    h_ref[...] = h
    onehot = (lax.broadcasted_iota(jnp.int32, (G, R), 0)
              == batch_ref[0]).astype(jnp.float32)
    part = jax.lax.dot_general(onehot, h, (((1,), (0,)), ((), ())),
                               preferred_element_type=jnp.float32)

    @pl.when(i == 0)
    def _():
        p_scr[...] = jnp.zeros_like(p_scr)

    p_scr[...] += part

    @pl.when(i == NBLK - 1)
    def _():
        z_ref[...] = jnp.dot(p_scr[...], wl_ref[...],
                             preferred_element_type=jnp.float32) + bl_ref[...]


_tc_final = pl.pallas_call(
    _tc_final_body,
    grid=(NBLK,),
    in_specs=[
        pl.BlockSpec((NC, R, DH), lambda i: (0, i, 0)),
        pl.BlockSpec((2, R, DH), lambda i: (0, i, 0)),
        pl.BlockSpec((R, 1), lambda i: (i, 0)),
        pl.BlockSpec((1, D), lambda i: (0, 0)),
        pl.BlockSpec((1, 1, R), lambda i: (i, 0, 0)),
        pl.BlockSpec((D, NCLS), lambda i: (0, 0)),
        pl.BlockSpec((1, NCLS), lambda i: (0, 0)),
    ],
    out_specs=[
        pl.BlockSpec((R, D), lambda i: (i, 0)),
        pl.BlockSpec((G, NCLS), lambda i: (0, 0)),
    ],
    out_shape=[
        jax.ShapeDtypeStruct((N, D), jnp.float32),
        jax.ShapeDtypeStruct((G, NCLS), jnp.float32),
    ],
    scratch_shapes=[pltpu.VMEM((G, D), jnp.float32)],
)


def kernel(x, edge_index, batch, W1, b1, W2, b2, W3, b3, W4, b4, Wl, bl):
    src = edge_index[0].astype(jnp.int32)
    dst = edge_index[1].astype(jnp.int32)
    src2 = jnp.concatenate([src, src + N])
    batch_row = batch.astype(jnp.int32).reshape(NBLK, 1, R)

    zrows = jnp.zeros((NPS, DH), jnp.float32)
    zdeg = jnp.zeros((NPS, DW), jnp.float32)
    ones = jnp.ones((C, DW), jnp.float32)

    deg = _sc_degree(dst, zdeg, ones).reshape(NC, NP, DW)
    degT = deg[:, :N, 0].T

    def prop(u):
        a = _sc_prop(u.reshape(2 * N, DH), src2, dst, zrows)
        return a.reshape(NC, NP, DH)

    u1, dis = _tc0(x, W1, degT)
    a1 = prop(u1)
    h1, u2 = _tc_mid(a1, u1, dis, b1.reshape(1, D), W2)
    a2 = prop(u2)
    h2, u3 = _tc_mid(a2, u2, dis, b2.reshape(1, D), W3)
    a3 = prop(u3)
    h3, u4 = _tc_mid(a3, u3, dis, b3.reshape(1, D), W4)
    a4 = prop(u4)
    h4, z = _tc_final(a4, u4, dis, b4.reshape(1, D), batch_row, Wl,
                      bl.reshape(1, NCLS))
    return (h1, h2, h3, h4, z)


# speedup vs baseline: 19.2694x; 3.1657x over previous
import functools

import jax
import jax.numpy as jnp
from jax import lax
from jax.experimental import pallas as pl
from jax.experimental.pallas import tpu as pltpu
from jax.experimental.pallas import tpu_sc as plsc

N = 10000
D = 128
DH = D // 2
E = 320000
G = 128
NCLS = 10

NC, NS = 2, 16
NW = NC * NS
EPW = E // NW
EPC = E // NS
C = 80
NCHUNK_DEG = EPW // C
NCHUNK = EPC // C

NP = 10240
NPS = NP // NS
DW = 16

_mesh = plsc.VectorSubcoreMesh(core_axis_name="c", subcore_axis_name="s")


NB_DEG = 8


@functools.partial(
    pl.kernel,
    out_type=jax.ShapeDtypeStruct((NC * NP, DW), jnp.float32),
    mesh=_mesh,
    scratch_types=[
        pltpu.VMEM((NCHUNK_DEG, C), jnp.int32),
        pltpu.VMEM((C, DW), jnp.float32),
        pltpu.VMEM((NPS, DW), jnp.float32),
        pltpu.VMEM_SHARED((NP, DW), jnp.float32),
        [pltpu.SemaphoreType.DMA] * NB_DEG,
    ],
    compiler_params=pltpu.CompilerParams(use_tc_tiling_on_sc=False),
)
def _sc_degree(dst_hbm, zeros_hbm, ones_hbm, out_hbm, didx, ones, bounce, acc,
               sems):
    c = lax.axis_index("c")
    s = lax.axis_index("s")
    wid = s * NC + c
    pltpu.sync_copy(dst_hbm.at[pl.ds(wid * NCHUNK_DEG, NCHUNK_DEG)], didx)
    pltpu.sync_copy(ones_hbm, ones)
    pltpu.sync_copy(zeros_hbm, bounce)
    pltpu.sync_copy(bounce, acc.at[pl.ds(s * NPS, NPS)])
    plsc.subcore_barrier()

    def body(k, carry):
        for b in range(NB_DEG):
            j = k * NB_DEG + b

            @pl.when(k > 0)
            def _():
                pltpu.make_async_copy(ones, acc.at[didx.at[j - NB_DEG]],
                                      sems[b]).wait()

            pltpu.async_copy(ones, acc.at[didx.at[j]], sems[b], add=True)
        return carry

    nk = NCHUNK_DEG // NB_DEG
    lax.fori_loop(0, nk, body, 0)
    for b in range(NB_DEG):
        j = (nk - 1) * NB_DEG + b
        pltpu.make_async_copy(ones, acc.at[didx.at[j]], sems[b]).wait()
    for j in range(nk * NB_DEG, NCHUNK_DEG):
        pltpu.sync_copy(ones, acc.at[didx.at[j]], add=True)
    plsc.subcore_barrier()
    pltpu.sync_copy(acc.at[pl.ds(s * NPS, NPS)], bounce)
    pltpu.sync_copy(bounce, out_hbm.at[pl.ds(c * NP + s * NPS, NPS)])


NB = 8
NK = NCHUNK // NB
NQ = NPS // C


@functools.partial(
    pl.kernel,
    out_type=jax.ShapeDtypeStruct((NC * NP, DH), jnp.float32),
    mesh=_mesh,
    scratch_types=[
        pltpu.VMEM((NCHUNK, C), jnp.int32),
        pltpu.VMEM((NCHUNK, C), jnp.int32),
        [pltpu.VMEM((C, DH), jnp.float32)] * NB,
        pltpu.VMEM_SHARED((NP, DH), jnp.float32),
        [pltpu.SemaphoreType.DMA] * NB,
        [pltpu.SemaphoreType.DMA] * NB,
    ],
    compiler_params=pltpu.CompilerParams(use_tc_tiling_on_sc=False),
)
def _sc_prop(u_hbm, src2_hbm, dst_hbm, zeros_hbm, out_hbm,
             sidx, didx, rows, acc, gsem, ssem):
    c = lax.axis_index("c")
    s = lax.axis_index("s")
    w = c * NS + s
    pltpu.sync_copy(src2_hbm.at[pl.ds(w * NCHUNK, NCHUNK)], sidx)
    pltpu.sync_copy(dst_hbm.at[pl.ds(s * NCHUNK, NCHUNK)], didx)
    pltpu.sync_copy(zeros_hbm, rows[0])
    descs = [
        pltpu.async_copy(rows[0], acc.at[pl.ds(s * NPS + q * C, C)], ssem[q])
        for q in range(NQ)
    ]
    for d in descs:
        d.wait()
    plsc.subcore_barrier()

    def gather(j, b):
        pltpu.async_copy(u_hbm.at[sidx.at[j]], rows[b], gsem[b])

    def gather_wait(j, b):
        pltpu.make_async_copy(u_hbm.at[sidx.at[j]], rows[b], gsem[b]).wait()

    def scatter(j, b):
        pltpu.async_copy(rows[b], acc.at[didx.at[j]], ssem[b], add=True)

    def scatter_wait(j, b):
        pltpu.make_async_copy(rows[b], acc.at[didx.at[j]], ssem[b]).wait()

    for b in range(NB):
        gather(b, b)

    def body(k, carry):
        base = k * NB
        for b in range(NB):
            gather_wait(base + b, b)
            scatter(base + b, b)
        for b in range(NB):

            @pl.when(k < NK - 1)
            def _():
                scatter_wait(base + b, b)
                gather(base + NB + b, b)

        return carry

    lax.fori_loop(0, NK, body, 0)
    for b in range(NB):
        scatter_wait((NK - 1) * NB + b, b)
    for j in range(NK * NB, NCHUNK):
        pltpu.async_copy(u_hbm.at[sidx.at[j]], rows[0], gsem[0])
        pltpu.make_async_copy(u_hbm.at[sidx.at[j]], rows[0], gsem[0]).wait()
        pltpu.sync_copy(rows[0], acc.at[didx.at[j]], add=True)
    plsc.subcore_barrier()
    odescs = []
    for q in range(NQ):
        pltpu.sync_copy(acc.at[pl.ds(s * NPS + q * C, C)], rows[q])
        odescs.append(pltpu.async_copy(
            rows[q], out_hbm.at[pl.ds(c * NP + s * NPS + q * C, C)], gsem[q]))
    for d in odescs:
        d.wait()


R = 400
NBLK = N // R


def _split(u):
    return u[:, :DH], u[:, DH:]


def _tc0_body(x_ref, w_ref, deg_ref, u_ref, dis_ref):
    d = deg_ref[:, 0:1] + deg_ref[:, 1:2] + 1.0
    dis = lax.rsqrt(d)
    dis_ref[...] = dis
    u = jnp.dot(x_ref[...], w_ref[...],
                preferred_element_type=jnp.float32) * dis
    u_ref[0], u_ref[1] = _split(u)


_tc0 = pl.pallas_call(
    _tc0_body,
    grid=(NBLK,),
    in_specs=[
        pl.BlockSpec((R, D), lambda i: (i, 0)),
        pl.BlockSpec((D, D), lambda i: (0, 0)),
        pl.BlockSpec((R, 2), lambda i: (i, 0)),
    ],
    out_specs=[
        pl.BlockSpec((2, R, DH), lambda i: (0, i, 0)),
        pl.BlockSpec((R, 1), lambda i: (i, 0)),
    ],
    out_shape=[
        jax.ShapeDtypeStruct((2, N, DH), jnp.float32),
        jax.ShapeDtypeStruct((N, 1), jnp.float32),
    ],
)


def _tc_mid_body(a_ref, u_ref, dis_ref, b_ref, w_ref, h_ref, un_ref):
    dis = dis_ref[...]
    agg = jnp.concatenate([a_ref[0] + u_ref[0], a_ref[1] + u_ref[1]], axis=1)
    h = jnp.maximum(agg * dis + b_ref[...], 0.0)
    h_ref[...] = h
    un = jnp.dot(h, w_ref[...], preferred_element_type=jnp.float32) * dis
    un_ref[0], un_ref[1] = _split(un)


_tc_mid = pl.pallas_call(
    _tc_mid_body,
    grid=(NBLK,),
    in_specs=[
        pl.BlockSpec((NC, R, DH), lambda i: (0, i, 0)),
        pl.BlockSpec((2, R, DH), lambda i: (0, i, 0)),
        pl.BlockSpec((R, 1), lambda i: (i, 0)),
        pl.BlockSpec((1, D), lambda i: (0, 0)),
        pl.BlockSpec((D, D), lambda i: (0, 0)),
    ],
    out_specs=[
        pl.BlockSpec((R, D), lambda i: (i, 0)),
        pl.BlockSpec((2, R, DH), lambda i: (0, i, 0)),
    ],
    out_shape=[
        jax.ShapeDtypeStruct((N, D), jnp.float32),
        jax.ShapeDtypeStruct((2, N, DH), jnp.float32),
    ],
)


def _tc_final_body(a_ref, u_ref, dis_ref, b_ref, batch_ref, wl_ref, bl_ref,
                   h_ref, z_ref, p_scr):
    i = pl.program_id(0)
    agg = jnp.concatenate([a_ref[0] + u_ref[0], a_ref[1] + u_ref[1]], axis=1)
    h = jnp.maximum(agg * dis_ref[...] + b_ref[...], 0.0)
    h_ref[...] = h
    onehot = (lax.broadcasted_iota(jnp.int32, (G, R), 0)
              == batch_ref[0]).astype(jnp.float32)
    part = jax.lax.dot_general(onehot, h, (((1,), (0,)), ((), ())),
                               preferred_element_type=jnp.float32)

    @pl.when(i == 0)
    def _():
        p_scr[...] = jnp.zeros_like(p_scr)

    p_scr[...] += part

    @pl.when(i == NBLK - 1)
    def _():
        z_ref[...] = jnp.dot(p_scr[...], wl_ref[...],
                             preferred_element_type=jnp.float32) + bl_ref[...]


_tc_final = pl.pallas_call(
    _tc_final_body,
    grid=(NBLK,),
    in_specs=[
        pl.BlockSpec((NC, R, DH), lambda i: (0, i, 0)),
        pl.BlockSpec((2, R, DH), lambda i: (0, i, 0)),
        pl.BlockSpec((R, 1), lambda i: (i, 0)),
        pl.BlockSpec((1, D), lambda i: (0, 0)),
        pl.BlockSpec((1, 1, R), lambda i: (i, 0, 0)),
        pl.BlockSpec((D, NCLS), lambda i: (0, 0)),
        pl.BlockSpec((1, NCLS), lambda i: (0, 0)),
    ],
    out_specs=[
        pl.BlockSpec((R, D), lambda i: (i, 0)),
        pl.BlockSpec((G, NCLS), lambda i: (0, 0)),
    ],
    out_shape=[
        jax.ShapeDtypeStruct((N, D), jnp.float32),
        jax.ShapeDtypeStruct((G, NCLS), jnp.float32),
    ],
    scratch_shapes=[pltpu.VMEM((G, D), jnp.float32)],
)


def kernel(x, edge_index, batch, W1, b1, W2, b2, W3, b3, W4, b4, Wl, bl):
    src = edge_index[0].astype(jnp.int32)
    dst = edge_index[1].astype(jnp.int32)
    src2 = jnp.concatenate([src, src + N]).reshape(NC * NS * NCHUNK, C)
    dst2 = dst.reshape(NS * NCHUNK, C)
    batch_row = batch.astype(jnp.int32).reshape(NBLK, 1, R)

    zrows = jnp.zeros((C, DH), jnp.float32)
    zdeg = jnp.zeros((NPS, DW), jnp.float32)
    ones = jnp.ones((C, DW), jnp.float32)

    deg = _sc_degree(dst2, zdeg, ones).reshape(NC, NP, DW)
    degT = deg[:, :N, 0].T

    def prop(u):
        a = _sc_prop(u.reshape(2 * N, DH), src2, dst2, zrows)
        return a.reshape(NC, NP, DH)

    u1, dis = _tc0(x, W1, degT)
    a1 = prop(u1)
    h1, u2 = _tc_mid(a1, u1, dis, b1.reshape(1, D), W2)
    a2 = prop(u2)
    h2, u3 = _tc_mid(a2, u2, dis, b2.reshape(1, D), W3)
    a3 = prop(u3)
    h3, u4 = _tc_mid(a3, u3, dis, b3.reshape(1, D), W4)
    a4 = prop(u4)
    h4, z = _tc_final(a4, u4, dis, b4.reshape(1, D), batch_row, Wl,
                      bl.reshape(1, NCLS))
    return (h1, h2, h3, h4, z)
